# baseline (speedup 1.0000x reference)
import math

import jax
import jax.numpy as jnp
from jax.experimental import pallas as pl
from jax.experimental.pallas import tpu as pltpu

_B, _S, _D, _E = 1, 2048, 768, 8
_H = 4 * _D
_TS = 256
_NI = _S // _TS
_SQ2PI = math.sqrt(2.0 / math.pi)


def _gelu(h):
    return 0.5 * h * (1.0 + jnp.tanh(_SQ2PI * (h + 0.044715 * (h * h * h))))


def _ffn_moe_kernel(x_ref, wfc_ref, bfc_ref, wproj_ref, bproj_ref,
                    wr_ref, br_ref, out_ref, mask_ref):
    e = pl.program_id(0)
    i = pl.program_id(1)

    @pl.when((e == 0) & (i == 0))
    def _router():
        xbar = jnp.mean(x_ref[0], axis=0, keepdims=True)
        scores = jnp.dot(xbar, wr_ref[...],
                         preferred_element_type=jnp.float32) + br_ref[...]
        mask_ref[...] = jax.nn.softmax(scores, axis=-1)

    x_tile = x_ref[0, pl.ds(i * _TS, _TS), :].astype(jnp.bfloat16)
    h = jnp.dot(x_tile, wfc_ref[0],
                preferred_element_type=jnp.float32) + bfc_ref[0]
    h = _gelu(h).astype(jnp.bfloat16)
    h2 = jnp.dot(h, wproj_ref[0],
                 preferred_element_type=jnp.float32) + bproj_ref[0]

    lane = jax.lax.broadcasted_iota(jnp.int32, (1, _E), 1)
    m = jnp.sum(jnp.where(lane == e, mask_ref[...], 0.0))

    @pl.when(e == 0)
    def _init():
        out_ref[0, pl.ds(i * _TS, _TS), :] = m * h2

    @pl.when(e > 0)
    def _acc():
        out_ref[0, pl.ds(i * _TS, _TS), :] += m * h2


def kernel(x, W_fc, b_fc, W_proj, b_proj, W_r, b_r):
    W_fc = W_fc.astype(jnp.bfloat16)
    W_proj = W_proj.astype(jnp.bfloat16)
    b_fc3 = b_fc.reshape(_E, 1, _H)
    b_proj3 = b_proj.reshape(_E, 1, _D)
    b_r2 = b_r.reshape(1, _E)
    out = pl.pallas_call(
        _ffn_moe_kernel,
        grid=(_E, _NI),
        in_specs=[
            pl.BlockSpec((1, _S, _D), lambda e, i: (0, 0, 0)),
            pl.BlockSpec((1, _D, _H), lambda e, i: (e, 0, 0)),
            pl.BlockSpec((1, 1, _H), lambda e, i: (e, 0, 0)),
            pl.BlockSpec((1, _H, _D), lambda e, i: (e, 0, 0)),
            pl.BlockSpec((1, 1, _D), lambda e, i: (e, 0, 0)),
            pl.BlockSpec((_D, _E), lambda e, i: (0, 0)),
            pl.BlockSpec((1, _E), lambda e, i: (0, 0)),
        ],
        out_specs=pl.BlockSpec((1, _S, _D), lambda e, i: (0, 0, 0)),
        out_shape=jax.ShapeDtypeStruct((_B, _S, _D), jnp.float32),
        scratch_shapes=[pltpu.VMEM((1, _E), jnp.float32)],
    )(x, W_fc, b_fc3, W_proj, b_proj3, W_r, b_r2)
    return out


# speedup vs baseline: 1.4176x; 1.4176x over previous
import math

import jax
import jax.numpy as jnp
from jax.experimental import pallas as pl
from jax.experimental.pallas import tpu as pltpu

_B, _S, _D, _E = 1, 2048, 768, 8
_H = 4 * _D
_TS = 512
_NI = _S // _TS
_SQ2PI = math.sqrt(2.0 / math.pi)


def _gelu(h):
    return 0.5 * h * (1.0 + jnp.tanh(_SQ2PI * (h + 0.044715 * (h * h * h))))


def _ffn_moe_kernel(x_ref, wfc_ref, bfc_ref, wproj_ref, bproj_ref,
                    wr_ref, br_ref, out_ref, mask_ref):
    e = pl.program_id(0)
    i = pl.program_id(1)

    @pl.when((e == 0) & (i == 0))
    def _router():
        xbar = jnp.mean(x_ref[0], axis=0, keepdims=True)
        scores = jnp.dot(xbar, wr_ref[...],
                         preferred_element_type=jnp.float32) + br_ref[...]
        mask_ref[...] = jax.nn.softmax(scores, axis=-1)

    x_tile = x_ref[0, pl.ds(i * _TS, _TS), :].astype(jnp.bfloat16)
    h = jnp.dot(x_tile, wfc_ref[0].astype(jnp.bfloat16),
                preferred_element_type=jnp.float32) + bfc_ref[0]
    h = _gelu(h).astype(jnp.bfloat16)
    h2 = jnp.dot(h, wproj_ref[0].astype(jnp.bfloat16),
                 preferred_element_type=jnp.float32) + bproj_ref[0]

    lane = jax.lax.broadcasted_iota(jnp.int32, (1, _E), 1)
    m = jnp.sum(jnp.where(lane == e, mask_ref[...], 0.0))

    @pl.when(e == 0)
    def _init():
        out_ref[0, pl.ds(i * _TS, _TS), :] = m * h2

    @pl.when(e > 0)
    def _acc():
        out_ref[0, pl.ds(i * _TS, _TS), :] += m * h2


def kernel(x, W_fc, b_fc, W_proj, b_proj, W_r, b_r):
    b_fc3 = b_fc.reshape(_E, 1, _H)
    b_proj3 = b_proj.reshape(_E, 1, _D)
    b_r2 = b_r.reshape(1, _E)
    out = pl.pallas_call(
        _ffn_moe_kernel,
        grid=(_E, _NI),
        in_specs=[
            pl.BlockSpec((1, _S, _D), lambda e, i: (0, 0, 0)),
            pl.BlockSpec((1, _D, _H), lambda e, i: (e, 0, 0)),
            pl.BlockSpec((1, 1, _H), lambda e, i: (e, 0, 0)),
            pl.BlockSpec((1, _H, _D), lambda e, i: (e, 0, 0)),
            pl.BlockSpec((1, 1, _D), lambda e, i: (e, 0, 0)),
            pl.BlockSpec((_D, _E), lambda e, i: (0, 0)),
            pl.BlockSpec((1, _E), lambda e, i: (0, 0)),
        ],
        out_specs=pl.BlockSpec((1, _S, _D), lambda e, i: (0, 0, 0)),
        out_shape=jax.ShapeDtypeStruct((_B, _S, _D), jnp.float32),
        scratch_shapes=[pltpu.VMEM((1, _E), jnp.float32)],
    )(x, W_fc, b_fc3, W_proj, b_proj3, W_r, b_r2)
    return out
